# Initial kernel scaffold; baseline (speedup 1.0000x reference)
import jax
import jax.numpy as jnp
from jax.experimental import pallas as pl


def kernel(inputs, kernel_weights, rows, cols, bias):
    raise NotImplementedError("write your pallas kernel here")


# speedup vs baseline: 6.4968x; 6.4968x over previous
import functools

import jax
import jax.numpy as jnp
from jax import lax
from jax.experimental import pallas as pl
from jax.experimental.pallas import tpu as pltpu
from jax.experimental.pallas import tpu_sc as plsc

UNITS = 4096
INPUT_DIM = 4096
BATCH = 128
CONN = 167772

NC = 2
NS = 16
LANES = 16
G = 128
CHUNKS_PER_TEC = -(-CONN // (NC * NS * G))
CONN_PER_TEC = CHUNKS_PER_TEC * G
C_PAD = NC * NS * CONN_PER_TEC
ROWS_PER_TEC = UNITS // NS
ZROWS = 64


_GATHER_DN = lax.GatherDimensionNumbers(
    offset_dims=(), collapsed_slice_dims=(0,), start_index_map=(0,)
)


def _vbroadcast(vec, k):
  idx = jnp.full((LANES, 1), k, jnp.int32)
  return lax.gather(
      vec, idx, _GATHER_DN, slice_sizes=(1,),
      mode=lax.GatherScatterMode.PROMISE_IN_BOUNDS,
  )


def _sc_spmm(xT, w_p, rows_p, cols_p):
  mesh = plsc.VectorSubcoreMesh(
      core_axis_name="c", subcore_axis_name="s", num_cores=NC, num_subcores=NS
  )

  @functools.partial(
      pl.kernel,
      mesh=mesh,
      out_type=jax.ShapeDtypeStruct((NC * UNITS, BATCH), jnp.float32),
      scratch_types=[
          pltpu.VMEM((G,), jnp.int32),
          pltpu.VMEM((G,), jnp.int32),
          pltpu.VMEM((G,), jnp.float32),
          pltpu.VMEM((G, BATCH), jnp.float32),
          pltpu.VMEM((ZROWS, BATCH), jnp.float32),
          pltpu.VMEM_SHARED((UNITS, BATCH), jnp.float32),
          pltpu.SemaphoreType.DMA,
      ],
  )
  def k(xT_hbm, w_hbm, rows_hbm, cols_hbm, out_hbm,
        rbuf, cbuf, wbuf, xbuf, zbuf, acc, sem):
    cid = lax.axis_index("c")
    sid = lax.axis_index("s")

    zero16 = jnp.zeros((LANES,), jnp.float32)

    def zrow(i, carry):
      for t in range(BATCH // LANES):
        zbuf[i, pl.ds(t * LANES, LANES)] = zero16
      return carry

    lax.fori_loop(0, ZROWS, zrow, 0)
    for q in range(ROWS_PER_TEC // ZROWS):
      pltpu.sync_copy(
          zbuf, acc.at[pl.ds(sid * ROWS_PER_TEC + q * ZROWS, ZROWS)]
      )
    plsc.subcore_barrier()

    base = (cid * NS + sid) * CONN_PER_TEC

    def chunk(i, carry):
      off = base + i * G
      pltpu.sync_copy(rows_hbm.at[pl.ds(off, G)], rbuf)
      pltpu.sync_copy(cols_hbm.at[pl.ds(off, G)], cbuf)
      pltpu.sync_copy(w_hbm.at[pl.ds(off, G)], wbuf)
      pltpu.async_copy(xT_hbm.at[rbuf], xbuf, sem).wait()

      def scale(g, c2):
        j0 = g * LANES
        wv16 = wbuf[pl.ds(j0, LANES)]
        for k in range(LANES):
          wk = _vbroadcast(wv16, k)
          row = j0 + k
          for t in range(BATCH // LANES):
            sl = pl.ds(t * LANES, LANES)
            xbuf[row, sl] = xbuf[row, sl] * wk
        return c2

      lax.fori_loop(0, G // LANES, scale, 0)
      pltpu.sync_copy(xbuf, acc.at[cbuf], add=True)
      return carry

    lax.fori_loop(0, CHUNKS_PER_TEC, chunk, 0)

    plsc.subcore_barrier()
    pltpu.sync_copy(
        acc.at[pl.ds(sid * ROWS_PER_TEC, ROWS_PER_TEC)],
        out_hbm.at[pl.ds(cid * UNITS + sid * ROWS_PER_TEC, ROWS_PER_TEC)],
    )

  return k(xT, w_p, rows_p, cols_p)


_UBLK = 512


def _combine_body(eye_ref, p_ref, b_ref, o_ref):
  p = p_ref[0] + p_ref[1]
  t = lax.dot_general(
      eye_ref[...], p, (((1,), (1,)), ((), ())),
      preferred_element_type=jnp.float32,
  )
  o_ref[...] = t + b_ref[...]


def _combine(partials, bias):
  eye = jnp.eye(BATCH, dtype=jnp.float32)
  return pl.pallas_call(
      _combine_body,
      grid=(UNITS // _UBLK,),
      in_specs=[
          pl.BlockSpec((BATCH, BATCH), lambda i: (0, 0)),
          pl.BlockSpec((NC, _UBLK, BATCH), lambda i: (0, i, 0)),
          pl.BlockSpec((1, _UBLK), lambda i: (0, i)),
      ],
      out_specs=pl.BlockSpec((BATCH, _UBLK), lambda i: (0, i)),
      out_shape=jax.ShapeDtypeStruct((BATCH, UNITS), jnp.float32),
  )(eye, partials, bias.reshape(1, UNITS))


def kernel(inputs, kernel_weights, rows, cols, bias):
  pad = C_PAD - CONN
  xT = inputs.T
  w_p = jnp.pad(kernel_weights, (0, pad))
  rows_p = jnp.pad(rows, (0, pad))
  cols_p = jnp.pad(cols, (0, pad))
  partials = _sc_spmm(xT, w_p, rows_p, cols_p)
  return _combine(partials.reshape(NC, UNITS, BATCH), bias)
